# Initial kernel scaffold; baseline (speedup 1.0000x reference)
import functools

import jax
import jax.numpy as jnp
import numpy as np
from jax import lax
from jax.experimental import pallas as pl
from jax.experimental.pallas import tpu as pltpu
from jax.experimental.pallas import tpu_sc as plsc

N = 10000
DIN = 128
DH = 256
HALF = DH // 2

NC = 2
NS = 16
CH = 128
ACC_R = 10008
TRASH = N

ROW_BLK = 2000
GRID = N // ROW_BLK

_BN_INV = float(1.0 / np.sqrt(1.0 + 1e-5))


def _pre_body(x_ref, g_ref, b_ref, w1_ref, b1_ref, wc_ref, out_ref):
    xb = x_ref[...]
    h = xb * (g_ref[...] * _BN_INV) + b_ref[...]
    h1 = lax.dot_general(h, w1_ref[...], (((1,), (1,)), ((), ())),
                         preferred_element_type=jnp.float32)
    h1 = jnp.maximum(h1 + b1_ref[...], 0.0)
    wc = wc_ref[...]
    out_ref[0] = lax.dot_general(h1, wc[:HALF], (((1,), (1,)), ((), ())),
                                 preferred_element_type=jnp.float32)
    out_ref[1] = lax.dot_general(h1, wc[HALF:], (((1,), (1,)), ((), ())),
                                 preferred_element_type=jnp.float32)


def _mid_body(a_ref, bias_ref, wc_ref, out_ref):
    b = bias_ref[...]
    h_lo = jnp.maximum(a_ref[0] + b[:, :HALF], 0.0)
    h_hi = jnp.maximum(a_ref[1] + b[:, HALF:], 0.0)
    wc = wc_ref[...]
    out_ref[0] = (
        lax.dot_general(h_lo, wc[:HALF, :HALF], (((1,), (1,)), ((), ())),
                        preferred_element_type=jnp.float32)
        + lax.dot_general(h_hi, wc[:HALF, HALF:], (((1,), (1,)), ((), ())),
                          preferred_element_type=jnp.float32))
    out_ref[1] = (
        lax.dot_general(h_lo, wc[HALF:, :HALF], (((1,), (1,)), ((), ())),
                        preferred_element_type=jnp.float32)
        + lax.dot_general(h_hi, wc[HALF:, HALF:], (((1,), (1,)), ((), ())),
                          preferred_element_type=jnp.float32))


def _post_body(a_ref, bias_ref, hw_ref, hb_ref, out_ref, acc_ref):
    i = pl.program_id(0)
    b = bias_ref[...]
    h_lo = jnp.maximum(a_ref[0] + b[:, :HALF], 0.0)
    h_hi = jnp.maximum(a_ref[1] + b[:, HALF:], 0.0)
    part = jnp.concatenate(
        [jnp.sum(h_lo, axis=0, keepdims=True),
         jnp.sum(h_hi, axis=0, keepdims=True)], axis=1)

    @pl.when(i == 0)
    def _():
        acc_ref[...] = part

    @pl.when(i > 0)
    def _():
        acc_ref[...] = acc_ref[...] + part

    @pl.when(i == GRID - 1)
    def _():
        g = acc_ref[...] * (1.0 / N)
        out_ref[...] = lax.dot_general(
            g, hw_ref[...], (((1,), (1,)), ((), ())),
            preferred_element_type=jnp.float32) + hb_ref[...]


_pre_call = pl.pallas_call(
    _pre_body,
    grid=(GRID,),
    in_specs=[
        pl.BlockSpec((ROW_BLK, DIN), lambda i: (i, 0)),
        pl.BlockSpec((1, DIN), lambda i: (0, 0)),
        pl.BlockSpec((1, DIN), lambda i: (0, 0)),
        pl.BlockSpec((DH, DIN), lambda i: (0, 0)),
        pl.BlockSpec((1, DH), lambda i: (0, 0)),
        pl.BlockSpec((DH, DH), lambda i: (0, 0)),
    ],
    out_specs=pl.BlockSpec((2, ROW_BLK, HALF), lambda i: (0, i, 0)),
    out_shape=jax.ShapeDtypeStruct((2, N, HALF), jnp.float32),
)

_mid_call = pl.pallas_call(
    _mid_body,
    grid=(GRID,),
    in_specs=[
        pl.BlockSpec((2, ROW_BLK, HALF), lambda i: (0, i, 0)),
        pl.BlockSpec((1, DH), lambda i: (0, 0)),
        pl.BlockSpec((DH, DH), lambda i: (0, 0)),
    ],
    out_specs=pl.BlockSpec((2, ROW_BLK, HALF), lambda i: (0, i, 0)),
    out_shape=jax.ShapeDtypeStruct((2, N, HALF), jnp.float32),
)

_post_call = pl.pallas_call(
    _post_body,
    grid=(GRID,),
    in_specs=[
        pl.BlockSpec((2, ROW_BLK, HALF), lambda i: (0, i, 0)),
        pl.BlockSpec((1, DH), lambda i: (0, 0)),
        pl.BlockSpec((1, DH), lambda i: (0, 0)),
        pl.BlockSpec((1, 1), lambda i: (0, 0)),
    ],
    out_specs=pl.BlockSpec((1, 1), lambda i: (0, 0)),
    out_shape=jax.ShapeDtypeStruct((1, 1), jnp.float32),
    scratch_shapes=[pltpu.VMEM((1, DH), jnp.float32)],
)


def _sc_agg_body(nchunk, xi_hbm, src_hbm, dst_hbm, out_hbm,
                 src_v, dst_v, buf0, buf1, acc, sem0, sem1):
    c = lax.axis_index("c")
    s = lax.axis_index("s")

    pltpu.sync_copy(src_hbm.at[s], src_v)
    pltpu.sync_copy(dst_hbm.at[s], dst_v)

    rows_per_tile = N // NS
    r0 = s * rows_per_tile
    pltpu.sync_copy(xi_hbm.at[c, pl.ds(r0, rows_per_tile)],
                    acc.at[pl.ds(r0, rows_per_tile)])
    plsc.subcore_barrier()

    table = xi_hbm.at[c]
    npair = nchunk // 2

    pltpu.make_async_copy(table.at[src_v.at[0]], buf0, sem0).start()
    pltpu.make_async_copy(table.at[src_v.at[1]], buf1, sem1).start()

    def body(i, carry):
        j0 = 2 * i
        pltpu.make_async_copy(table.at[src_v.at[j0]], buf0, sem0).wait()
        pltpu.sync_copy(buf0, acc.at[dst_v.at[j0]], add=True)

        @pl.when(i < npair - 1)
        def _():
            pltpu.make_async_copy(table.at[src_v.at[j0 + 2]], buf0,
                                  sem0).start()

        pltpu.make_async_copy(table.at[src_v.at[j0 + 1]], buf1, sem1).wait()
        pltpu.sync_copy(buf1, acc.at[dst_v.at[j0 + 1]], add=True)

        @pl.when(i < npair - 1)
        def _():
            pltpu.make_async_copy(table.at[src_v.at[j0 + 3]], buf1,
                                  sem1).start()

        return carry

    lax.fori_loop(0, npair, body, 0)

    plsc.subcore_barrier()
    pltpu.sync_copy(acc.at[pl.ds(r0, rows_per_tile)],
                    out_hbm.at[c, pl.ds(r0, rows_per_tile)])


def _make_sc_agg(nchunk):
    mesh = plsc.VectorSubcoreMesh(core_axis_name="c", subcore_axis_name="s")
    return pl.kernel(
        functools.partial(_sc_agg_body, nchunk),
        out_type=jax.ShapeDtypeStruct((2, N, HALF), jnp.float32),
        mesh=mesh,
        scratch_types=[
            pltpu.VMEM((nchunk, CH), jnp.int32),
            pltpu.VMEM((nchunk, CH), jnp.int32),
            pltpu.VMEM((CH, HALF), jnp.float32),
            pltpu.VMEM((CH, HALF), jnp.float32),
            pltpu.VMEM_SHARED((ACC_R, HALF), jnp.float32),
            pltpu.SemaphoreType.DMA,
            pltpu.SemaphoreType.DMA,
        ],
    )


@jax.jit
def kernel(x, edge_index, bn_gamma, bn_beta, lin1_W, lin1_b,
           conv1_Win, conv1_bias, conv2_Win, conv2_bias, head_W, head_b):
    E = edge_index.shape[1]
    per_tile = -(-E // (NS * 2 * CH)) * 2 * CH
    nchunk = per_tile // CH
    pad = NS * per_tile - E

    src = jnp.concatenate(
        [edge_index[0], jnp.zeros((pad,), jnp.int32)]).reshape(NS, nchunk, CH)
    dst = jnp.concatenate(
        [edge_index[1], jnp.full((pad,), TRASH, jnp.int32)]).reshape(
            NS, nchunk, CH)

    gamma = bn_gamma.reshape(1, DIN)
    beta = bn_beta.reshape(1, DIN)
    b1 = lin1_b.reshape(1, DH)
    cb1 = conv1_bias.reshape(1, DH)
    cb2 = conv2_bias.reshape(1, DH)
    hw = head_W.reshape(1, DH)
    hb = head_b.reshape(1, 1)

    sc_agg = _make_sc_agg(nchunk)

    xi1 = _pre_call(x, gamma, beta, lin1_W, b1, conv1_Win)
    agg1 = sc_agg(xi1, src, dst)
    xi2 = _mid_call(agg1, cb1, conv2_Win)
    agg2 = sc_agg(xi2, src, dst)
    out = _post_call(agg2, cb2, hw, hb)
    return out


# speedup vs baseline: 4.2588x; 4.2588x over previous
import functools

import jax
import jax.numpy as jnp
import numpy as np
from jax import lax
from jax.experimental import pallas as pl
from jax.experimental.pallas import tpu as pltpu
from jax.experimental.pallas import tpu_sc as plsc

N = 10000
DIN = 128
DH = 256
HALF = DH // 2

NC = 2
NS = 16
CH = 128
ACC_R = 10008
TRASH = N

ROW_BLK = 2000
GRID = N // ROW_BLK

_BN_INV = float(1.0 / np.sqrt(1.0 + 1e-5))


def _pre_body(x_ref, g_ref, b_ref, w1_ref, b1_ref, wc_ref, out_ref):
    xb = x_ref[...]
    h = xb * (g_ref[...] * _BN_INV) + b_ref[...]
    h1 = lax.dot_general(h, w1_ref[...], (((1,), (1,)), ((), ())),
                         preferred_element_type=jnp.float32)
    h1 = jnp.maximum(h1 + b1_ref[...], 0.0)
    wc = wc_ref[...]
    out_ref[0] = lax.dot_general(h1, wc[:HALF], (((1,), (1,)), ((), ())),
                                 preferred_element_type=jnp.float32)
    out_ref[1] = lax.dot_general(h1, wc[HALF:], (((1,), (1,)), ((), ())),
                                 preferred_element_type=jnp.float32)


def _mid_body(a_ref, bias_ref, wc_ref, out_ref):
    b = bias_ref[...]
    h_lo = jnp.maximum(a_ref[0] + b[:, :HALF], 0.0)
    h_hi = jnp.maximum(a_ref[1] + b[:, HALF:], 0.0)
    wc = wc_ref[...]
    out_ref[0] = (
        lax.dot_general(h_lo, wc[:HALF, :HALF], (((1,), (1,)), ((), ())),
                        preferred_element_type=jnp.float32)
        + lax.dot_general(h_hi, wc[:HALF, HALF:], (((1,), (1,)), ((), ())),
                          preferred_element_type=jnp.float32))
    out_ref[1] = (
        lax.dot_general(h_lo, wc[HALF:, :HALF], (((1,), (1,)), ((), ())),
                        preferred_element_type=jnp.float32)
        + lax.dot_general(h_hi, wc[HALF:, HALF:], (((1,), (1,)), ((), ())),
                          preferred_element_type=jnp.float32))


def _post_body(a_ref, bias_ref, hw_ref, hb_ref, out_ref, acc_ref):
    i = pl.program_id(0)
    b = bias_ref[...]
    h_lo = jnp.maximum(a_ref[0] + b[:, :HALF], 0.0)
    h_hi = jnp.maximum(a_ref[1] + b[:, HALF:], 0.0)
    part = jnp.concatenate(
        [jnp.sum(h_lo, axis=0, keepdims=True),
         jnp.sum(h_hi, axis=0, keepdims=True)], axis=1)

    @pl.when(i == 0)
    def _():
        acc_ref[...] = part

    @pl.when(i > 0)
    def _():
        acc_ref[...] = acc_ref[...] + part

    @pl.when(i == GRID - 1)
    def _():
        g = acc_ref[...] * (1.0 / N)
        out_ref[...] = (jnp.sum(g * hw_ref[...], axis=1, keepdims=True)
                        + hb_ref[...])


_pre_call = pl.pallas_call(
    _pre_body,
    grid=(GRID,),
    in_specs=[
        pl.BlockSpec((ROW_BLK, DIN), lambda i: (i, 0)),
        pl.BlockSpec((1, DIN), lambda i: (0, 0)),
        pl.BlockSpec((1, DIN), lambda i: (0, 0)),
        pl.BlockSpec((DH, DIN), lambda i: (0, 0)),
        pl.BlockSpec((1, DH), lambda i: (0, 0)),
        pl.BlockSpec((DH, DH), lambda i: (0, 0)),
    ],
    out_specs=pl.BlockSpec((2, ROW_BLK, HALF), lambda i: (0, i, 0)),
    out_shape=jax.ShapeDtypeStruct((2, N, HALF), jnp.float32),
)

_mid_call = pl.pallas_call(
    _mid_body,
    grid=(GRID,),
    in_specs=[
        pl.BlockSpec((2, ROW_BLK, HALF), lambda i: (0, i, 0)),
        pl.BlockSpec((1, DH), lambda i: (0, 0)),
        pl.BlockSpec((DH, DH), lambda i: (0, 0)),
    ],
    out_specs=pl.BlockSpec((2, ROW_BLK, HALF), lambda i: (0, i, 0)),
    out_shape=jax.ShapeDtypeStruct((2, N, HALF), jnp.float32),
)

_post_call = pl.pallas_call(
    _post_body,
    grid=(GRID,),
    in_specs=[
        pl.BlockSpec((2, ROW_BLK, HALF), lambda i: (0, i, 0)),
        pl.BlockSpec((1, DH), lambda i: (0, 0)),
        pl.BlockSpec((1, DH), lambda i: (0, 0)),
        pl.BlockSpec((1, 1), lambda i: (0, 0)),
    ],
    out_specs=pl.BlockSpec((1, 1), lambda i: (0, 0)),
    out_shape=jax.ShapeDtypeStruct((1, 1), jnp.float32),
    scratch_shapes=[pltpu.VMEM((1, DH), jnp.float32)],
)


SUP = 8


def _sc_agg_body(nchunk, xi_hbm, src_hbm, dst_hbm, out_hbm,
                 sis, dis, r0b, r1b, acc, semia, semib, semg0, semg1):
    c = lax.axis_index("c")
    s = lax.axis_index("s")
    nsp = nchunk // (2 * SUP)

    rows_per_tile = (N // NS) // 8 * 8
    tail = N - rows_per_tile * NS
    r0 = s * rows_per_tile
    pltpu.sync_copy(xi_hbm.at[c, pl.ds(r0, rows_per_tile)],
                    acc.at[pl.ds(r0, rows_per_tile)])

    @pl.when(s == 0)
    def _():
        pltpu.sync_copy(xi_hbm.at[c, pl.ds(N - tail, tail)],
                        acc.at[pl.ds(N - tail, tail)])

    plsc.subcore_barrier()

    table = xi_hbm.at[c]
    rbuf = (r0b, r1b)
    gsem = (semg0, semg1)

    pltpu.sync_copy(src_hbm.at[s, pl.ds(0, SUP)], sis.at[0])
    pltpu.sync_copy(dst_hbm.at[s, pl.ds(0, SUP)], dis.at[0])
    pltpu.make_async_copy(src_hbm.at[s, pl.ds(SUP, SUP)], sis.at[1],
                          semib).start()
    pltpu.make_async_copy(dst_hbm.at[s, pl.ds(SUP, SUP)], dis.at[1],
                          semib).start()
    pltpu.make_async_copy(table.at[sis.at[0, 0]], r0b, semg0).start()
    pltpu.make_async_copy(table.at[sis.at[0, 1]], r1b, semg1).start()

    def body(i, carry):
        not_last = i < nsp - 1
        for k in range(2 * SUP):
            b, kk, p = k // SUP, k % SUP, k % 2
            pltpu.make_async_copy(table.at[sis.at[b, kk]], rbuf[p],
                                  gsem[p]).wait()
            pltpu.sync_copy(rbuf[p], acc.at[dis.at[b, kk]], add=True)

            if k == SUP - 1:
                @pl.when(not_last)
                def _():
                    off = (2 * i + 2) * SUP
                    pltpu.make_async_copy(src_hbm.at[s, pl.ds(off, SUP)],
                                          sis.at[0], semia).start()
                    pltpu.make_async_copy(dst_hbm.at[s, pl.ds(off, SUP)],
                                          dis.at[0], semia).start()
            if k == 2 * SUP - 1:
                @pl.when(not_last)
                def _():
                    off = (2 * i + 3) * SUP
                    pltpu.make_async_copy(src_hbm.at[s, pl.ds(off, SUP)],
                                          sis.at[1], semib).start()
                    pltpu.make_async_copy(dst_hbm.at[s, pl.ds(off, SUP)],
                                          dis.at[1], semib).start()

            if k < 2 * SUP - 2:
                b2, kk2 = (k + 2) // SUP, (k + 2) % SUP
                if k == SUP - 2:
                    pltpu.make_async_copy(src_hbm.at[s, pl.ds(0, SUP)],
                                          sis.at[1], semib).wait()
                    pltpu.make_async_copy(dst_hbm.at[s, pl.ds(0, SUP)],
                                          dis.at[1], semib).wait()
                pltpu.make_async_copy(table.at[sis.at[b2, kk2]], rbuf[p],
                                      gsem[p]).start()
            else:
                @pl.when(not_last)
                def _():
                    if k == 2 * SUP - 2:
                        pltpu.make_async_copy(src_hbm.at[s, pl.ds(0, SUP)],
                                              sis.at[0], semia).wait()
                        pltpu.make_async_copy(dst_hbm.at[s, pl.ds(0, SUP)],
                                              dis.at[0], semia).wait()
                    pltpu.make_async_copy(table.at[sis.at[0, k % 2]],
                                          rbuf[p], gsem[p]).start()
        return carry

    lax.fori_loop(0, nsp, body, 0)

    plsc.subcore_barrier()
    pltpu.sync_copy(acc.at[pl.ds(r0, rows_per_tile)],
                    out_hbm.at[c, pl.ds(r0, rows_per_tile)])

    @pl.when(s == 0)
    def _():
        pltpu.sync_copy(acc.at[pl.ds(N - tail, tail)],
                        out_hbm.at[c, pl.ds(N - tail, tail)])


def _make_sc_agg(nchunk):
    mesh = plsc.VectorSubcoreMesh(core_axis_name="c", subcore_axis_name="s")
    return pl.kernel(
        functools.partial(_sc_agg_body, nchunk),
        out_type=jax.ShapeDtypeStruct((2, N, HALF), jnp.float32),
        mesh=mesh,
        scratch_types=[
            pltpu.VMEM((2, SUP, CH), jnp.int32),
            pltpu.VMEM((2, SUP, CH), jnp.int32),
            pltpu.VMEM((CH, HALF), jnp.float32),
            pltpu.VMEM((CH, HALF), jnp.float32),
            pltpu.VMEM_SHARED((ACC_R, HALF), jnp.float32),
            pltpu.SemaphoreType.DMA,
            pltpu.SemaphoreType.DMA,
            pltpu.SemaphoreType.DMA,
            pltpu.SemaphoreType.DMA,
        ],
    )


@jax.jit
def kernel(x, edge_index, bn_gamma, bn_beta, lin1_W, lin1_b,
           conv1_Win, conv1_bias, conv2_Win, conv2_bias, head_W, head_b):
    E = edge_index.shape[1]
    gran = 2 * SUP * CH
    per_tile = -(-E // (NS * gran)) * gran
    nchunk = per_tile // CH
    pad = NS * per_tile - E

    src = jnp.concatenate(
        [edge_index[0], jnp.zeros((pad,), jnp.int32)]).reshape(NS, nchunk, CH)
    dst = jnp.concatenate(
        [edge_index[1], jnp.full((pad,), TRASH, jnp.int32)]).reshape(
            NS, nchunk, CH)

    gamma = bn_gamma.reshape(1, DIN)
    beta = bn_beta.reshape(1, DIN)
    b1 = lin1_b.reshape(1, DH)
    cb1 = conv1_bias.reshape(1, DH)
    cb2 = conv2_bias.reshape(1, DH)
    hw = head_W.reshape(1, DH)
    hb = head_b.reshape(1, 1)

    sc_agg = _make_sc_agg(nchunk)

    xi1 = _pre_call(x, gamma, beta, lin1_W, b1, conv1_Win)
    agg1 = sc_agg(xi1, src, dst)
    xi2 = _mid_call(agg1, cb1, conv2_Win)
    agg2 = sc_agg(xi2, src, dst)
    out = _post_call(agg2, cb2, hw, hb)
    return out
